# Initial kernel scaffold; baseline (speedup 1.0000x reference)
import jax
import jax.numpy as jnp
from jax.experimental import pallas as pl


def kernel(memory, spatial_shapes, W_proj, b_proj, g_proj, be_proj, W_score, b_score, W1, b1, W2, b2, W3, b3, Wm, bm, gm, bem):
    raise NotImplementedError("write your pallas kernel here")


# speedup vs baseline: 1.8342x; 1.8342x over previous
import functools

import jax
import jax.numpy as jnp
import numpy as np
from jax.experimental import pallas as pl
from jax.experimental.pallas import tpu as pltpu

_HIDDEN = 256
_NUM_CLASSES = 80
_NUM_QUERIES = 300
_ANCHOR_EPS = 0.01
_SHAPES = [(80, 80), (40, 40), (20, 20)]
_B = 16
_S = 8400
_SBLK = 840
_QBLK = 1200


def _anchors_and_mask(spatial_shapes):
    anchors = []
    masks = []
    for level, (h, w) in enumerate(_SHAPES):
        gy, gx = jnp.meshgrid(jnp.arange(h, dtype=jnp.float32),
                              jnp.arange(w, dtype=jnp.float32), indexing='ij')
        grid_xy = jnp.stack((gx, gy), axis=-1)
        valid_wh = spatial_shapes[level, ::-1].astype(jnp.float32)
        centers = (grid_xy + 0.5) / valid_wh
        wh = jnp.ones_like(centers) * (0.05 * 2.0 ** level)
        anchor = jnp.concatenate((centers, wh), axis=-1).reshape(h * w, 4)
        mask = jnp.all((anchor > _ANCHOR_EPS) & (anchor < 1.0 - _ANCHOR_EPS),
                       axis=-1, keepdims=True)
        a = jnp.clip(anchor, 1e-4, 1.0 - 1e-4)
        anchors.append(jnp.log(a / (1.0 - a)))
        masks.append(mask)
    return jnp.concatenate(anchors, axis=0), jnp.concatenate(masks, axis=0)


def _ln(x, g, b):
    mu = jnp.mean(x, axis=-1, keepdims=True)
    xc = x - mu
    var = jnp.mean(xc * xc, axis=-1, keepdims=True)
    return xc / jnp.sqrt(var + 1e-5) * g + b


def _stage1_body(mem_ref, mask_ref, wp_ref, bp_ref, g_ref, be_ref,
                 ws_ref, bs_ref, masked_ref, cls_ref):
    x = mem_ref[0] * mask_ref[...]
    masked_ref[0] = x
    enc = _ln(jnp.dot(x, wp_ref[...], preferred_element_type=jnp.float32)
              + bp_ref[...], g_ref[...], be_ref[...])
    sc = jnp.dot(enc, ws_ref[...], preferred_element_type=jnp.float32) + bs_ref[...]
    cls_ref[0, 0, :] = jnp.max(sc, axis=-1)


def _stage1(memory, maskf, W_proj, b_proj, g_proj, be_proj, W_score, b_score):
    n_s = _S // _SBLK
    grid = (_B, n_s)
    return pl.pallas_call(
        _stage1_body,
        grid=grid,
        in_specs=[
            pl.BlockSpec((1, _SBLK, _HIDDEN), lambda b, s: (b, s, 0)),
            pl.BlockSpec((_SBLK, 1), lambda b, s: (s, 0)),
            pl.BlockSpec((_HIDDEN, _HIDDEN), lambda b, s: (0, 0)),
            pl.BlockSpec((1, _HIDDEN), lambda b, s: (0, 0)),
            pl.BlockSpec((1, _HIDDEN), lambda b, s: (0, 0)),
            pl.BlockSpec((1, _HIDDEN), lambda b, s: (0, 0)),
            pl.BlockSpec((_HIDDEN, _NUM_CLASSES), lambda b, s: (0, 0)),
            pl.BlockSpec((1, _NUM_CLASSES), lambda b, s: (0, 0)),
        ],
        out_specs=[
            pl.BlockSpec((1, _SBLK, _HIDDEN), lambda b, s: (b, s, 0)),
            pl.BlockSpec((1, 1, _SBLK), lambda b, s: (b * n_s + s, 0, 0)),
        ],
        out_shape=[
            jax.ShapeDtypeStruct((_B, _S, _HIDDEN), jnp.float32),
            jax.ShapeDtypeStruct((_B * n_s, 1, _SBLK), jnp.float32),
        ],
    )(memory, maskf, W_proj, b_proj, g_proj, be_proj, W_score, b_score)


def _topk_body(cls_ref, idx_ref, a_ref):
    a_ref[...] = cls_ref[...]
    iota = jax.lax.broadcasted_iota(jnp.int32, (_B, _S), 1)
    qiota = jax.lax.broadcasted_iota(jnp.int32, (_B, _NUM_QUERIES), 1)

    def step(i, acc):
        a = a_ref[...]
        m = jnp.max(a, axis=1, keepdims=True)
        cand = jnp.where(a == m, iota, _S)
        amin = jnp.min(cand, axis=1, keepdims=True)
        acc = jnp.where(qiota == i, amin, acc)
        a_ref[...] = jnp.where(cand == amin, -jnp.inf, a)
        return acc

    idx_ref[...] = jax.lax.fori_loop(
        0, _NUM_QUERIES, step, jnp.zeros((_B, _NUM_QUERIES), jnp.int32))


def _topk(cls_max):
    return pl.pallas_call(
        _topk_body,
        grid=(1,),
        in_specs=[pl.BlockSpec((_B, _S), lambda i: (0, 0))],
        out_specs=pl.BlockSpec((_B, _NUM_QUERIES), lambda i: (0, 0)),
        out_shape=jax.ShapeDtypeStruct((_B, _NUM_QUERIES), jnp.int32),
        scratch_shapes=[pltpu.VMEM((_B, _S), jnp.float32)],
    )(cls_max)


def _stage4_body(x_ref, anch_ref, wp_ref, bp_ref, g_ref, be_ref, ws_ref, bs_ref,
                 w1_ref, b1_ref, w2_ref, b2_ref, w3_ref, b3_ref,
                 tgt_ref, logits_ref, refp_ref, boxes_ref):
    x = x_ref[...]
    enc = _ln(jnp.dot(x, wp_ref[...], preferred_element_type=jnp.float32)
              + bp_ref[...], g_ref[...], be_ref[...])
    tgt_ref[...] = enc
    logits_ref[...] = jnp.dot(enc, ws_ref[...], preferred_element_type=jnp.float32) + bs_ref[...]
    h = jax.nn.relu(jnp.dot(enc, w1_ref[...], preferred_element_type=jnp.float32) + b1_ref[...])
    h = jax.nn.relu(jnp.dot(h, w2_ref[...], preferred_element_type=jnp.float32) + b2_ref[...])
    bu = jnp.dot(h, w3_ref[...], preferred_element_type=jnp.float32) + b3_ref[...] + anch_ref[...]
    refp_ref[...] = bu
    boxes_ref[...] = jax.nn.sigmoid(bu)


def _stage4(x, anch, W_proj, b_proj, g_proj, be_proj, W_score, b_score,
            W1, b1, W2, b2, W3, b3):
    n = _B * _NUM_QUERIES
    grid = (n // _QBLK,)
    full = lambda r, c: pl.BlockSpec((r, c), lambda i: (0, 0))
    return pl.pallas_call(
        _stage4_body,
        grid=grid,
        in_specs=[
            pl.BlockSpec((_QBLK, _HIDDEN), lambda i: (i, 0)),
            pl.BlockSpec((_QBLK, 4), lambda i: (i, 0)),
            full(_HIDDEN, _HIDDEN), full(1, _HIDDEN), full(1, _HIDDEN), full(1, _HIDDEN),
            full(_HIDDEN, _NUM_CLASSES), full(1, _NUM_CLASSES),
            full(_HIDDEN, _HIDDEN), full(1, _HIDDEN),
            full(_HIDDEN, _HIDDEN), full(1, _HIDDEN),
            full(_HIDDEN, 4), full(1, 4),
        ],
        out_specs=[
            pl.BlockSpec((_QBLK, _HIDDEN), lambda i: (i, 0)),
            pl.BlockSpec((_QBLK, _NUM_CLASSES), lambda i: (i, 0)),
            pl.BlockSpec((_QBLK, 4), lambda i: (i, 0)),
            pl.BlockSpec((_QBLK, 4), lambda i: (i, 0)),
        ],
        out_shape=[
            jax.ShapeDtypeStruct((n, _HIDDEN), jnp.float32),
            jax.ShapeDtypeStruct((n, _NUM_CLASSES), jnp.float32),
            jax.ShapeDtypeStruct((n, 4), jnp.float32),
            jax.ShapeDtypeStruct((n, 4), jnp.float32),
        ],
    )(x, anch, W_proj, b_proj, g_proj, be_proj, W_score, b_score,
      W1, b1, W2, b2, W3, b3)


def kernel(memory, spatial_shapes, W_proj, b_proj, g_proj, be_proj, W_score,
           b_score, W1, b1, W2, b2, W3, b3, Wm, bm, gm, bem):
    anchors, valid_mask = _anchors_and_mask(spatial_shapes)
    maskf = valid_mask.astype(jnp.float32)

    masked_memory, cls_blk = _stage1(
        memory, maskf, W_proj, b_proj.reshape(1, -1), g_proj.reshape(1, -1),
        be_proj.reshape(1, -1), W_score, b_score.reshape(1, -1))

    topk = _topk(cls_blk.reshape(_B, _S))

    rows = jnp.take_along_axis(masked_memory, topk[:, :, None], axis=1)
    anch_g = jnp.take(anchors, topk.reshape(-1), axis=0)

    target, logits, refp, boxes = _stage4(
        rows.reshape(_B * _NUM_QUERIES, _HIDDEN), anch_g,
        W_proj, b_proj.reshape(1, -1), g_proj.reshape(1, -1),
        be_proj.reshape(1, -1), W_score, b_score.reshape(1, -1),
        W1, b1.reshape(1, -1), W2, b2.reshape(1, -1), W3, b3.reshape(1, -1))

    shp = (_B, _NUM_QUERIES)
    return (target.reshape(*shp, _HIDDEN),
            refp.reshape(*shp, 4),
            boxes.reshape(*shp, 4),
            logits.reshape(*shp, _NUM_CLASSES),
            masked_memory)


# speedup vs baseline: 2.4009x; 1.3090x over previous
import functools

import jax
import jax.numpy as jnp
import numpy as np
from jax.experimental import pallas as pl
from jax.experimental.pallas import tpu as pltpu

_HIDDEN = 256
_NUM_CLASSES = 80
_NUM_QUERIES = 300
_ANCHOR_EPS = 0.01
_SHAPES = [(80, 80), (40, 40), (20, 20)]
_B = 16
_S = 8400
_SBLK = 1680
_QBLK = 1200


def _anchors_and_mask(spatial_shapes):
    anchors = []
    masks = []
    for level, (h, w) in enumerate(_SHAPES):
        gy, gx = jnp.meshgrid(jnp.arange(h, dtype=jnp.float32),
                              jnp.arange(w, dtype=jnp.float32), indexing='ij')
        grid_xy = jnp.stack((gx, gy), axis=-1)
        valid_wh = spatial_shapes[level, ::-1].astype(jnp.float32)
        centers = (grid_xy + 0.5) / valid_wh
        wh = jnp.ones_like(centers) * (0.05 * 2.0 ** level)
        anchor = jnp.concatenate((centers, wh), axis=-1).reshape(h * w, 4)
        mask = jnp.all((anchor > _ANCHOR_EPS) & (anchor < 1.0 - _ANCHOR_EPS),
                       axis=-1, keepdims=True)
        a = jnp.clip(anchor, 1e-4, 1.0 - 1e-4)
        anchors.append(jnp.log(a / (1.0 - a)))
        masks.append(mask)
    return jnp.concatenate(anchors, axis=0), jnp.concatenate(masks, axis=0)


def _ln(x, g, b):
    mu = jnp.mean(x, axis=-1, keepdims=True)
    xc = x - mu
    var = jnp.mean(xc * xc, axis=-1, keepdims=True)
    return xc / jnp.sqrt(var + 1e-5) * g + b


def _stage1_body(mem_ref, mask_ref, wp_ref, bp_ref, g_ref, be_ref,
                 ws_ref, bs_ref, masked_ref, cls_ref):
    x = mem_ref[0] * mask_ref[...]
    masked_ref[0] = x
    enc = _ln(jnp.dot(x, wp_ref[...], preferred_element_type=jnp.float32)
              + bp_ref[...], g_ref[...], be_ref[...])
    sc = jnp.dot(enc, ws_ref[...], preferred_element_type=jnp.float32) + bs_ref[...]
    cls_ref[0, 0, :] = jnp.max(sc, axis=-1)


def _stage1(memory, maskf, W_proj, b_proj, g_proj, be_proj, W_score, b_score):
    n_s = _S // _SBLK
    grid = (_B, n_s)
    return pl.pallas_call(
        _stage1_body,
        grid=grid,
        in_specs=[
            pl.BlockSpec((1, _SBLK, _HIDDEN), lambda b, s: (b, s, 0)),
            pl.BlockSpec((_SBLK, 1), lambda b, s: (s, 0)),
            pl.BlockSpec((_HIDDEN, _HIDDEN), lambda b, s: (0, 0)),
            pl.BlockSpec((1, _HIDDEN), lambda b, s: (0, 0)),
            pl.BlockSpec((1, _HIDDEN), lambda b, s: (0, 0)),
            pl.BlockSpec((1, _HIDDEN), lambda b, s: (0, 0)),
            pl.BlockSpec((_HIDDEN, _NUM_CLASSES), lambda b, s: (0, 0)),
            pl.BlockSpec((1, _NUM_CLASSES), lambda b, s: (0, 0)),
        ],
        out_specs=[
            pl.BlockSpec((1, _SBLK, _HIDDEN), lambda b, s: (b, s, 0)),
            pl.BlockSpec((1, 1, _SBLK), lambda b, s: (b * n_s + s, 0, 0)),
        ],
        out_shape=[
            jax.ShapeDtypeStruct((_B, _S, _HIDDEN), jnp.float32),
            jax.ShapeDtypeStruct((_B * n_s, 1, _SBLK), jnp.float32),
        ],
    )(memory, maskf, W_proj, b_proj, g_proj, be_proj, W_score, b_score)


def _topk_body(cls_ref, idx_ref, a_ref):
    a_ref[...] = cls_ref[...]
    iota = jax.lax.broadcasted_iota(jnp.int32, (_B, _S), 1)
    qiota = jax.lax.broadcasted_iota(jnp.int32, (_B, _NUM_QUERIES), 1)

    def step(i, acc):
        a = a_ref[...]
        m = jnp.max(a, axis=1, keepdims=True)
        cand = jnp.where(a == m, iota, _S)
        amin = jnp.min(cand, axis=1, keepdims=True)
        acc = jnp.where(qiota == i, amin, acc)
        a_ref[...] = jnp.where(cand == amin, -jnp.inf, a)
        return acc

    idx_ref[...] = jax.lax.fori_loop(
        0, _NUM_QUERIES, step, jnp.zeros((_B, _NUM_QUERIES), jnp.int32))


def _topk(cls_max):
    return pl.pallas_call(
        _topk_body,
        grid=(1,),
        in_specs=[pl.BlockSpec((_B, _S), lambda i: (0, 0))],
        out_specs=pl.BlockSpec((_B, _NUM_QUERIES), lambda i: (0, 0)),
        out_shape=jax.ShapeDtypeStruct((_B, _NUM_QUERIES), jnp.int32),
        scratch_shapes=[pltpu.VMEM((_B, _S), jnp.float32)],
    )(cls_max)


def _stage4_body(x_ref, anch_ref, wp_ref, bp_ref, g_ref, be_ref, ws_ref, bs_ref,
                 w1_ref, b1_ref, w2_ref, b2_ref, w3_ref, b3_ref,
                 tgt_ref, logits_ref, refp_ref, boxes_ref):
    x = x_ref[...]
    enc = _ln(jnp.dot(x, wp_ref[...], preferred_element_type=jnp.float32)
              + bp_ref[...], g_ref[...], be_ref[...])
    tgt_ref[...] = enc
    logits_ref[...] = jnp.dot(enc, ws_ref[...], preferred_element_type=jnp.float32) + bs_ref[...]
    h = jax.nn.relu(jnp.dot(enc, w1_ref[...], preferred_element_type=jnp.float32) + b1_ref[...])
    h = jax.nn.relu(jnp.dot(h, w2_ref[...], preferred_element_type=jnp.float32) + b2_ref[...])
    bu = jnp.dot(h, w3_ref[...], preferred_element_type=jnp.float32) + b3_ref[...] + anch_ref[...]
    refp_ref[...] = bu
    boxes_ref[...] = jax.nn.sigmoid(bu)


def _stage4(x, anch, W_proj, b_proj, g_proj, be_proj, W_score, b_score,
            W1, b1, W2, b2, W3, b3):
    n = _B * _NUM_QUERIES
    grid = (n // _QBLK,)
    full = lambda r, c: pl.BlockSpec((r, c), lambda i: (0, 0))
    return pl.pallas_call(
        _stage4_body,
        grid=grid,
        in_specs=[
            pl.BlockSpec((_QBLK, _HIDDEN), lambda i: (i, 0)),
            pl.BlockSpec((_QBLK, 4), lambda i: (i, 0)),
            full(_HIDDEN, _HIDDEN), full(1, _HIDDEN), full(1, _HIDDEN), full(1, _HIDDEN),
            full(_HIDDEN, _NUM_CLASSES), full(1, _NUM_CLASSES),
            full(_HIDDEN, _HIDDEN), full(1, _HIDDEN),
            full(_HIDDEN, _HIDDEN), full(1, _HIDDEN),
            full(_HIDDEN, 4), full(1, 4),
        ],
        out_specs=[
            pl.BlockSpec((_QBLK, _HIDDEN), lambda i: (i, 0)),
            pl.BlockSpec((_QBLK, _NUM_CLASSES), lambda i: (i, 0)),
            pl.BlockSpec((_QBLK, 4), lambda i: (i, 0)),
            pl.BlockSpec((_QBLK, 4), lambda i: (i, 0)),
        ],
        out_shape=[
            jax.ShapeDtypeStruct((n, _HIDDEN), jnp.float32),
            jax.ShapeDtypeStruct((n, _NUM_CLASSES), jnp.float32),
            jax.ShapeDtypeStruct((n, 4), jnp.float32),
            jax.ShapeDtypeStruct((n, 4), jnp.float32),
        ],
    )(x, anch, W_proj, b_proj, g_proj, be_proj, W_score, b_score,
      W1, b1, W2, b2, W3, b3)


def kernel(memory, spatial_shapes, W_proj, b_proj, g_proj, be_proj, W_score,
           b_score, W1, b1, W2, b2, W3, b3, Wm, bm, gm, bem):
    anchors, valid_mask = _anchors_and_mask(spatial_shapes)
    maskf = valid_mask.astype(jnp.float32)

    masked_memory, cls_blk = _stage1(
        memory, maskf, W_proj, b_proj.reshape(1, -1), g_proj.reshape(1, -1),
        be_proj.reshape(1, -1), W_score, b_score.reshape(1, -1))

    topk = _topk(cls_blk.reshape(_B, _S))

    rows = jnp.take_along_axis(masked_memory, topk[:, :, None], axis=1)
    anch_g = jnp.take(anchors, topk.reshape(-1), axis=0)

    target, logits, refp, boxes = _stage4(
        rows.reshape(_B * _NUM_QUERIES, _HIDDEN), anch_g,
        W_proj, b_proj.reshape(1, -1), g_proj.reshape(1, -1),
        be_proj.reshape(1, -1), W_score, b_score.reshape(1, -1),
        W1, b1.reshape(1, -1), W2, b2.reshape(1, -1), W3, b3.reshape(1, -1))

    shp = (_B, _NUM_QUERIES)
    return (target.reshape(*shp, _HIDDEN),
            refp.reshape(*shp, 4),
            boxes.reshape(*shp, 4),
            logits.reshape(*shp, _NUM_CLASSES),
            masked_memory)


# speedup vs baseline: 2.7817x; 1.1586x over previous
import functools

import jax
import jax.numpy as jnp
import numpy as np
from jax.experimental import pallas as pl
from jax.experimental.pallas import tpu as pltpu
from jax import lax
from jax.experimental.pallas import tpu_sc as plsc

_HIDDEN = 256
_NUM_CLASSES = 80
_NUM_QUERIES = 300
_ANCHOR_EPS = 0.01
_SHAPES = [(80, 80), (40, 40), (20, 20)]
_B = 16
_S = 8400
_SBLK = 1680
_QBLK = 1200


def _anchors_and_mask(spatial_shapes):
    anchors = []
    masks = []
    for level, (h, w) in enumerate(_SHAPES):
        gy, gx = jnp.meshgrid(jnp.arange(h, dtype=jnp.float32),
                              jnp.arange(w, dtype=jnp.float32), indexing='ij')
        grid_xy = jnp.stack((gx, gy), axis=-1)
        valid_wh = spatial_shapes[level, ::-1].astype(jnp.float32)
        centers = (grid_xy + 0.5) / valid_wh
        wh = jnp.ones_like(centers) * (0.05 * 2.0 ** level)
        anchor = jnp.concatenate((centers, wh), axis=-1).reshape(h * w, 4)
        mask = jnp.all((anchor > _ANCHOR_EPS) & (anchor < 1.0 - _ANCHOR_EPS),
                       axis=-1, keepdims=True)
        a = jnp.clip(anchor, 1e-4, 1.0 - 1e-4)
        anchors.append(jnp.log(a / (1.0 - a)))
        masks.append(mask)
    return jnp.concatenate(anchors, axis=0), jnp.concatenate(masks, axis=0)


def _ln(x, g, b):
    mu = jnp.mean(x, axis=-1, keepdims=True)
    xc = x - mu
    var = jnp.mean(xc * xc, axis=-1, keepdims=True)
    return xc / jnp.sqrt(var + 1e-5) * g + b


def _stage1_body(mem_ref, mask_ref, wp_ref, bp_ref, g_ref, be_ref,
                 ws_ref, bs_ref, masked_ref, cls_ref):
    x = mem_ref[0] * mask_ref[...]
    masked_ref[0] = x
    enc = _ln(jnp.dot(x, wp_ref[...], preferred_element_type=jnp.float32)
              + bp_ref[...], g_ref[...], be_ref[...])
    sc = jnp.dot(enc, ws_ref[...], preferred_element_type=jnp.float32) + bs_ref[...]
    cls_ref[0, 0, :] = jnp.max(sc, axis=-1)


def _stage1(memory, maskf, W_proj, b_proj, g_proj, be_proj, W_score, b_score):
    n_s = _S // _SBLK
    grid = (_B, n_s)
    return pl.pallas_call(
        _stage1_body,
        grid=grid,
        in_specs=[
            pl.BlockSpec((1, _SBLK, _HIDDEN), lambda b, s: (b, s, 0)),
            pl.BlockSpec((_SBLK, 1), lambda b, s: (s, 0)),
            pl.BlockSpec((_HIDDEN, _HIDDEN), lambda b, s: (0, 0)),
            pl.BlockSpec((1, _HIDDEN), lambda b, s: (0, 0)),
            pl.BlockSpec((1, _HIDDEN), lambda b, s: (0, 0)),
            pl.BlockSpec((1, _HIDDEN), lambda b, s: (0, 0)),
            pl.BlockSpec((_HIDDEN, _NUM_CLASSES), lambda b, s: (0, 0)),
            pl.BlockSpec((1, _NUM_CLASSES), lambda b, s: (0, 0)),
        ],
        out_specs=[
            pl.BlockSpec((1, _SBLK, _HIDDEN), lambda b, s: (b, s, 0)),
            pl.BlockSpec((1, 1, _SBLK), lambda b, s: (b * n_s + s, 0, 0)),
        ],
        out_shape=[
            jax.ShapeDtypeStruct((_B, _S, _HIDDEN), jnp.float32),
            jax.ShapeDtypeStruct((_B * n_s, 1, _SBLK), jnp.float32),
        ],
    )(memory, maskf, W_proj, b_proj, g_proj, be_proj, W_score, b_score)


_NSEL = 304
_PADIDX = 0x40000000


def _keyify(v):
    u = lax.bitcast_convert_type(v, jnp.uint32)
    neg = (u >> jnp.uint32(31)) == jnp.uint32(1)
    return jnp.where(neg, ~u, u | jnp.uint32(0x80000000))


def _digit(k, lvl):
    return ((k >> jnp.uint32(24 - 8 * lvl)) & jnp.uint32(0xFF)).astype(jnp.int32)


def _sum16(v_i32):
    return jnp.sum(v_i32)


def _sc_body(cls_hbm, mem_hbm, anch_hbm, rows_out, anchg_out,
             keys_v, ck1, ci1, ck2, ci2, hist, sel_k, sel_i, ord_v,
             flat_a, flat_b, orda_v, ordb_v, rows_v, anch_v, sem):
    cid = lax.axis_index("c")
    sid = lax.axis_index("s")
    wid = sid * 2 + cid

    @pl.when(wid < _B)
    def _work():
        b = wid
        lanes = lax.iota(jnp.int32, 16)
        ones = jnp.ones((16,), jnp.int32)
        zeros_i = jnp.zeros((16,), jnp.int32)

        pltpu.sync_copy(cls_hbm.at[b], keys_v)

        def init_sel(j, _):
            sel_k[pl.ds(j * 16, 16)] = jnp.zeros((16,), jnp.int32)
            sel_i[pl.ds(j * 16, 16)] = _PADIDX + j * 16 + lanes
            return 0
        lax.fori_loop(0, _NSEL // 16, init_sel, 0)

        def zero_hist(j, _):
            hist[pl.ds(j * 16, 16)] = zeros_i
            return 0

        def find_bin(k_rem):
            def stp(t, carry):
                cum, bsel, above, found = carry
                j = 255 - t
                s = _sum16(hist[pl.ds(j * 16, 16)])
                ncum = cum + s
                hit = jnp.logical_and(found == 0, ncum >= k_rem)
                bsel = jnp.where(hit, j, bsel)
                above = jnp.where(hit, cum, above)
                found = jnp.where(hit, jnp.int32(1), found)
                return (ncum, bsel, above, found)
            _, bsel, above, _ = lax.fori_loop(
                0, 256, stp,
                (jnp.int32(0), jnp.int32(0), jnp.int32(0), jnp.int32(0)))
            return bsel, above

        def run_level(lvl, ck_a, ci_a, ck_b, ci_b, n_cand, sel_n, k_rem, last):
            nv = (n_cand + 15) // 16

            def load_kv(j):
                if lvl == 0:
                    return _keyify(keys_v[pl.ds(j * 16, 16)]), j * 16 + lanes
                return lax.bitcast_convert_type(ck_a[pl.ds(j * 16, 16)], jnp.uint32), ci_a[pl.ds(j * 16, 16)]

            if lvl > 0:
                base = pl.multiple_of(nv * 16 - 16, 16)
                m_pad = (base + lanes) >= n_cand
                ck_a[pl.ds(base, 16)] = jnp.where(
                    m_pad, jnp.int32(0), ck_a[pl.ds(base, 16)])
                ci_a[pl.ds(base, 16)] = jnp.where(
                    m_pad, _PADIDX + base + lanes, ci_a[pl.ds(base, 16)])

            lax.fori_loop(0, 256, zero_hist, 0)

            def lh(j, _):
                k, _i = load_kv(j)
                d = _digit(k, lvl)
                plsc.addupdate_scatter(hist, [d * 16 + lanes], ones)
                return 0
            lax.fori_loop(0, nv, lh, 0)

            bl, above = find_bin(k_rem)
            k_rem2 = k_rem - above

            def le(j, carry):
                sel_n, cn, eqtaken = carry
                k, idx = load_kv(j)
                d = _digit(k, lvl)
                m_ab = d > bl
                m_eq = d == bl
                if last:
                    c_eq = plsc.cumsum(m_eq.astype(jnp.int32))
                    m_take = jnp.logical_or(
                        m_ab, jnp.logical_and(m_eq, (eqtaken + c_eq) <= k_rem2))
                    c_t = plsc.cumsum(m_take.astype(jnp.int32))
                    plsc.store_scatter(sel_k, [sel_n + c_t - 1], lax.bitcast_convert_type(k, jnp.int32), mask=m_take)
                    plsc.store_scatter(sel_i, [sel_n + c_t - 1], idx, mask=m_take)
                    return (sel_n + _sum16(m_take.astype(jnp.int32)), cn,
                            eqtaken + _sum16(m_eq.astype(jnp.int32)))
                c_ab = plsc.cumsum(m_ab.astype(jnp.int32))
                c_eq = plsc.cumsum(m_eq.astype(jnp.int32))
                plsc.store_scatter(sel_k, [sel_n + c_ab - 1], lax.bitcast_convert_type(k, jnp.int32), mask=m_ab)
                plsc.store_scatter(sel_i, [sel_n + c_ab - 1], idx, mask=m_ab)
                plsc.store_scatter(ck_b, [cn + c_eq - 1], lax.bitcast_convert_type(k, jnp.int32), mask=m_eq)
                plsc.store_scatter(ci_b, [cn + c_eq - 1], idx, mask=m_eq)
                return (sel_n + _sum16(m_ab.astype(jnp.int32)),
                        cn + _sum16(m_eq.astype(jnp.int32)), eqtaken)

            sel_n, n2, _ = lax.fori_loop(
                0, nv, le, (sel_n, jnp.int32(0), jnp.int32(0)))
            return n2, sel_n, k_rem2

        n, sel_n, k_rem = run_level(0, None, None, ck1, ci1,
                                    jnp.int32(_S), jnp.int32(0),
                                    jnp.int32(_NUM_QUERIES), False)
        n, sel_n, k_rem = run_level(1, ck1, ci1, ck2, ci2, n, sel_n, k_rem, False)
        n, sel_n, k_rem = run_level(2, ck2, ci2, ck1, ci1, n, sel_n, k_rem, False)
        n, sel_n, k_rem = run_level(3, ck1, ci1, None, None, n, sel_n, k_rem, True)

        def rank_one(e, _):
            eb = pl.multiple_of((e // 16) * 16, 16)
            el = e - eb
            kvec = sel_k[pl.ds(eb, 16)]
            ivec = sel_i[pl.ds(eb, 16)]
            msel = lanes == el
            ke = jnp.sum(jnp.where(msel, kvec, 0))
            ie = jnp.sum(jnp.where(msel, ivec, 0))
            keu = lax.bitcast_convert_type(ke, jnp.uint32)

            def acc_j(j, acc):
                kv = lax.bitcast_convert_type(sel_k[pl.ds(j * 16, 16)], jnp.uint32)
                iv = sel_i[pl.ds(j * 16, 16)]
                gt = (kv > keu).astype(jnp.int32)
                eqlt = jnp.logical_and(kv == keu, iv < ie).astype(jnp.int32)
                return acc + gt + eqlt
            acc = lax.fori_loop(0, _NSEL // 16, acc_j, zeros_i)
            r = _sum16(acc)
            plsc.store_scatter(ord_v, [jnp.full((16,), r, jnp.int32)],
                               jnp.full((16,), ie, jnp.int32),
                               mask=lanes == 0)
            return 0
        lax.fori_loop(0, _NSEL, rank_one, 0)

        tailb = _NSEL - 16
        tv = ord_v[pl.ds(tailb, 16)]
        ord_v[pl.ds(tailb, 16)] = jnp.where((tailb + lanes) >= _NUM_QUERIES,
                                            jnp.int32(0), tv)

        for j in range(_NSEL // 16):
            ov = ord_v[pl.ds(j * 16, 16)]
            if j < 10:
                flat_a[pl.ds(j * 16, 16)] = ov + b * _S
                orda_v[pl.ds(j * 16, 16)] = ov
            else:
                flat_b[pl.ds((j - 10) * 16, 16)] = ov + b * _S
                ordb_v[pl.ds((j - 10) * 16, 16)] = ov

        pltpu.async_copy(mem_hbm.at[flat_a], rows_v, sem).wait()
        pltpu.sync_copy(rows_v, rows_out.at[b, pl.ds(0, 160)])
        pltpu.async_copy(mem_hbm.at[flat_b], rows_v.at[pl.ds(0, 144)], sem).wait()
        pltpu.sync_copy(rows_v.at[pl.ds(0, 144)], rows_out.at[b, pl.ds(160, 144)])
        pltpu.async_copy(anch_hbm.at[orda_v], anch_v, sem).wait()
        pltpu.sync_copy(anch_v, anchg_out.at[b, pl.ds(0, 160)])
        pltpu.async_copy(anch_hbm.at[ordb_v], anch_v.at[pl.ds(0, 144)], sem).wait()
        pltpu.sync_copy(anch_v.at[pl.ds(0, 144)], anchg_out.at[b, pl.ds(160, 144)])


def sc_topk_gather(cls_max, mem_flat, anchors):
    mesh = plsc.VectorSubcoreMesh(core_axis_name="c", subcore_axis_name="s", num_cores=2, num_subcores=16)
    f = pl.kernel(
        _sc_body,
        out_type=[
            jax.ShapeDtypeStruct((_B, _NSEL, _HIDDEN), jnp.float32),
            jax.ShapeDtypeStruct((_B, _NSEL, 128), jnp.float32),
        ],
        mesh=mesh,
        compiler_params=pltpu.CompilerParams(needs_layout_passes=False),
        scratch_types=[
            pltpu.VMEM((_S,), jnp.float32),
            pltpu.VMEM((_S,), jnp.int32),
            pltpu.VMEM((_S,), jnp.int32),
            pltpu.VMEM((_S,), jnp.int32),
            pltpu.VMEM((_S,), jnp.int32),
            pltpu.VMEM((4096,), jnp.int32),
            pltpu.VMEM((_NSEL,), jnp.int32),
            pltpu.VMEM((_NSEL,), jnp.int32),
            pltpu.VMEM((_NSEL,), jnp.int32),
            pltpu.VMEM((160,), jnp.int32),
            pltpu.VMEM((144,), jnp.int32),
            pltpu.VMEM((160,), jnp.int32),
            pltpu.VMEM((144,), jnp.int32),
            pltpu.VMEM((160, _HIDDEN), jnp.float32),
            pltpu.VMEM((160, 128), jnp.float32),
            pltpu.SemaphoreType.DMA,
        ],
    )
    return f(cls_max, mem_flat, anchors)


def _stage4_body(x_ref, anch_ref, wp_ref, bp_ref, g_ref, be_ref, ws_ref, bs_ref,
                 w1_ref, b1_ref, w2_ref, b2_ref, w3_ref, b3_ref,
                 tgt_ref, logits_ref, refp_ref, boxes_ref):
    x = x_ref[...]
    enc = _ln(jnp.dot(x, wp_ref[...], preferred_element_type=jnp.float32)
              + bp_ref[...], g_ref[...], be_ref[...])
    tgt_ref[...] = enc
    logits_ref[...] = jnp.dot(enc, ws_ref[...], preferred_element_type=jnp.float32) + bs_ref[...]
    h = jax.nn.relu(jnp.dot(enc, w1_ref[...], preferred_element_type=jnp.float32) + b1_ref[...])
    h = jax.nn.relu(jnp.dot(h, w2_ref[...], preferred_element_type=jnp.float32) + b2_ref[...])
    bu = jnp.dot(h, w3_ref[...], preferred_element_type=jnp.float32) + b3_ref[...] + anch_ref[...]
    refp_ref[...] = bu
    boxes_ref[...] = jax.nn.sigmoid(bu)


def _stage4(x, anch, W_proj, b_proj, g_proj, be_proj, W_score, b_score,
            W1, b1, W2, b2, W3, b3):
    n = _B * _NUM_QUERIES
    grid = (n // _QBLK,)
    full = lambda r, c: pl.BlockSpec((r, c), lambda i: (0, 0))
    return pl.pallas_call(
        _stage4_body,
        grid=grid,
        in_specs=[
            pl.BlockSpec((_QBLK, _HIDDEN), lambda i: (i, 0)),
            pl.BlockSpec((_QBLK, 4), lambda i: (i, 0)),
            full(_HIDDEN, _HIDDEN), full(1, _HIDDEN), full(1, _HIDDEN), full(1, _HIDDEN),
            full(_HIDDEN, _NUM_CLASSES), full(1, _NUM_CLASSES),
            full(_HIDDEN, _HIDDEN), full(1, _HIDDEN),
            full(_HIDDEN, _HIDDEN), full(1, _HIDDEN),
            full(_HIDDEN, 4), full(1, 4),
        ],
        out_specs=[
            pl.BlockSpec((_QBLK, _HIDDEN), lambda i: (i, 0)),
            pl.BlockSpec((_QBLK, _NUM_CLASSES), lambda i: (i, 0)),
            pl.BlockSpec((_QBLK, 4), lambda i: (i, 0)),
            pl.BlockSpec((_QBLK, 4), lambda i: (i, 0)),
        ],
        out_shape=[
            jax.ShapeDtypeStruct((n, _HIDDEN), jnp.float32),
            jax.ShapeDtypeStruct((n, _NUM_CLASSES), jnp.float32),
            jax.ShapeDtypeStruct((n, 4), jnp.float32),
            jax.ShapeDtypeStruct((n, 4), jnp.float32),
        ],
    )(x, anch, W_proj, b_proj, g_proj, be_proj, W_score, b_score,
      W1, b1, W2, b2, W3, b3)


def kernel(memory, spatial_shapes, W_proj, b_proj, g_proj, be_proj, W_score,
           b_score, W1, b1, W2, b2, W3, b3, Wm, bm, gm, bem):
    anchors, valid_mask = _anchors_and_mask(spatial_shapes)
    maskf = valid_mask.astype(jnp.float32)

    masked_memory, cls_blk = _stage1(
        memory, maskf, W_proj, b_proj.reshape(1, -1), g_proj.reshape(1, -1),
        be_proj.reshape(1, -1), W_score, b_score.reshape(1, -1))

    anchors_pad = jnp.zeros((_S, 128), jnp.float32).at[:, :4].set(anchors)
    rows_pad, anch_pad = sc_topk_gather(
        cls_blk.reshape(_B, _S),
        masked_memory.reshape(_B * _S, _HIDDEN), anchors_pad)
    rows = rows_pad[:, :_NUM_QUERIES]
    anch_g = anch_pad[:, :_NUM_QUERIES, :4]

    target, logits, refp, boxes = _stage4(
        rows.reshape(_B * _NUM_QUERIES, _HIDDEN),
        anch_g.reshape(_B * _NUM_QUERIES, 4),
        W_proj, b_proj.reshape(1, -1), g_proj.reshape(1, -1),
        be_proj.reshape(1, -1), W_score, b_score.reshape(1, -1),
        W1, b1.reshape(1, -1), W2, b2.reshape(1, -1), W3, b3.reshape(1, -1))

    shp = (_B, _NUM_QUERIES)
    return (target.reshape(*shp, _HIDDEN),
            refp.reshape(*shp, 4),
            boxes.reshape(*shp, 4),
            logits.reshape(*shp, _NUM_CLASSES),
            masked_memory)
